# baseline (speedup 1.0000x reference)
import functools

import jax
import jax.numpy as jnp
from jax import lax
from jax.experimental import pallas as pl
from jax.experimental.pallas import tpu as pltpu
from jax.experimental.pallas import tpu_sc as plsc

VOCAB = 100000
DIM = 128
BATCH = 1024

V_TILE = 4096


@functools.cache
def _make_gather():
    info = plsc.get_sparse_core_info()
    nc, ns = info.num_cores, info.num_subcores
    nw = nc * ns
    b_per_w = BATCH // nw
    mesh = plsc.VectorSubcoreMesh(core_axis_name="c", subcore_axis_name="s")

    @functools.partial(
        pl.kernel,
        mesh=mesh,
        out_type=jax.ShapeDtypeStruct((BATCH, DIM), jnp.float32),
        scratch_types=[
            pltpu.VMEM((b_per_w,), jnp.int32),
            pltpu.VMEM((b_per_w, DIM), jnp.float32),
            pltpu.SemaphoreType.DMA,
        ],
    )
    def gather(idx_hbm, table_hbm, out_hbm, idx_v, rows_v, sem):
        wid = lax.axis_index("s") * nc + lax.axis_index("c")
        base = wid * b_per_w
        pltpu.sync_copy(idx_hbm.at[pl.ds(base, b_per_w)], idx_v)
        pltpu.async_copy(table_hbm.at[idx_v], rows_v, sem).wait()
        pltpu.sync_copy(rows_v, out_hbm.at[pl.ds(base, b_per_w)])

    return gather


def _proj_kernel(emb_ref, w_ref, b_ref, out_ref, embt_ref):
    @pl.when((pl.program_id(0) == 0) & (pl.program_id(1) == 0))
    def _():
        embt_ref[...] = emb_ref[...].T

    j = pl.program_id(1)
    out_ref[...] = lax.dot_general(
        w_ref[...], embt_ref[:, pl.ds(j * (BATCH // 2), BATCH // 2)],
        dimension_numbers=(((1,), (0,)), ((), ())),
        preferred_element_type=jnp.float32,
    ) + b_ref[...].T


def _project(emb, lin_w, lin_b2d):
    return pl.pallas_call(
        _proj_kernel,
        grid=(pl.cdiv(VOCAB, V_TILE), 2),
        in_specs=[
            pl.BlockSpec((BATCH, DIM), lambda i, j: (0, 0)),
            pl.BlockSpec((V_TILE, DIM), lambda i, j: (i, 0)),
            pl.BlockSpec((1, V_TILE), lambda i, j: (0, i)),
        ],
        out_specs=pl.BlockSpec((V_TILE, BATCH // 2), lambda i, j: (i, j)),
        out_shape=jax.ShapeDtypeStruct((VOCAB, BATCH), jnp.float32),
        scratch_shapes=[pltpu.VMEM((DIM, BATCH), jnp.float32)],
    )(emb, lin_w, lin_b2d)


def kernel(input_ids, emb_table, lin_w, lin_b):
    emb = _make_gather()(input_ids, emb_table)
    out_t = _project(emb, lin_w, lin_b.reshape(1, VOCAB))
    return out_t.T


# speedup vs baseline: 1.0433x; 1.0433x over previous
import functools

import jax
import jax.numpy as jnp
from jax import lax
from jax.experimental import pallas as pl
from jax.experimental.pallas import tpu as pltpu
from jax.experimental.pallas import tpu_sc as plsc

VOCAB = 100000
DIM = 128
BATCH = 1024

V_TILE = 5000


@functools.cache
def _make_gather():
    info = plsc.get_sparse_core_info()
    nc, ns = info.num_cores, info.num_subcores
    nw = nc * ns
    b_per_w = BATCH // nw
    mesh = plsc.VectorSubcoreMesh(core_axis_name="c", subcore_axis_name="s")

    @functools.partial(
        pl.kernel,
        mesh=mesh,
        out_type=jax.ShapeDtypeStruct((BATCH, DIM), jnp.float32),
        scratch_types=[
            pltpu.VMEM((b_per_w,), jnp.int32),
            pltpu.VMEM((b_per_w, DIM), jnp.float32),
            pltpu.SemaphoreType.DMA,
        ],
    )
    def gather(idx_hbm, table_hbm, out_hbm, idx_v, rows_v, sem):
        wid = lax.axis_index("s") * nc + lax.axis_index("c")
        base = wid * b_per_w
        pltpu.sync_copy(idx_hbm.at[pl.ds(base, b_per_w)], idx_v)
        pltpu.async_copy(table_hbm.at[idx_v], rows_v, sem).wait()
        pltpu.sync_copy(rows_v, out_hbm.at[pl.ds(base, b_per_w)])

    return gather


def _proj_kernel(emb_ref, w_ref, b_ref, out_ref, embt_ref):
    @pl.when(pl.program_id(0) == 0)
    def _():
        embt_ref[...] = emb_ref[...].T

    out_ref[...] = lax.dot_general(
        w_ref[...], embt_ref[...],
        dimension_numbers=(((1,), (0,)), ((), ())),
        preferred_element_type=jnp.float32,
    ) + b_ref[0].T


def _project(emb, lin_w, lin_b2d):
    return pl.pallas_call(
        _proj_kernel,
        grid=(pl.cdiv(VOCAB, V_TILE),),
        in_specs=[
            pl.BlockSpec((BATCH, DIM), lambda i: (0, 0)),
            pl.BlockSpec((V_TILE, DIM), lambda i: (i, 0)),
            pl.BlockSpec((1, 1, V_TILE), lambda i: (i, 0, 0)),
        ],
        out_specs=pl.BlockSpec((V_TILE, BATCH), lambda i: (i, 0)),
        out_shape=jax.ShapeDtypeStruct((VOCAB, BATCH), jnp.float32),
        scratch_shapes=[pltpu.VMEM((DIM, BATCH), jnp.float32)],
    )(emb, lin_w, lin_b2d)


def kernel(input_ids, emb_table, lin_w, lin_b):
    emb = _make_gather()(input_ids, emb_table)
    out_t = _project(emb, lin_w, lin_b.reshape(VOCAB // V_TILE, 1, V_TILE))
    return out_t.T


# speedup vs baseline: 1.0563x; 1.0125x over previous
import functools

import jax
import jax.numpy as jnp
from jax import lax
from jax.experimental import pallas as pl
from jax.experimental.pallas import tpu as pltpu
from jax.experimental.pallas import tpu_sc as plsc

VOCAB = 100000
DIM = 128
BATCH = 1024

V_TILE = 4000


@functools.cache
def _make_gather():
    info = plsc.get_sparse_core_info()
    nc, ns = info.num_cores, info.num_subcores
    nw = nc * ns
    b_per_w = BATCH // nw
    mesh = plsc.VectorSubcoreMesh(core_axis_name="c", subcore_axis_name="s")

    @functools.partial(
        pl.kernel,
        mesh=mesh,
        out_type=jax.ShapeDtypeStruct((BATCH, DIM), jnp.float32),
        scratch_types=[
            pltpu.VMEM((b_per_w,), jnp.int32),
            pltpu.VMEM((b_per_w, DIM), jnp.float32),
            pltpu.SemaphoreType.DMA,
        ],
    )
    def gather(idx_hbm, table_hbm, out_hbm, idx_v, rows_v, sem):
        wid = lax.axis_index("s") * nc + lax.axis_index("c")
        base = wid * b_per_w
        pltpu.sync_copy(idx_hbm.at[pl.ds(base, b_per_w)], idx_v)
        pltpu.async_copy(table_hbm.at[idx_v], rows_v, sem).wait()
        pltpu.sync_copy(rows_v, out_hbm.at[pl.ds(base, b_per_w)])

    return gather


def _proj_kernel(emb_ref, w_ref, b_ref, out_ref, embt_ref):
    @pl.when(pl.program_id(0) == 0)
    def _():
        embt_ref[...] = emb_ref[...].T

    out_ref[...] = lax.dot_general(
        w_ref[...], embt_ref[...],
        dimension_numbers=(((1,), (0,)), ((), ())),
        preferred_element_type=jnp.float32,
    ) + b_ref[0].T


def _project(emb, lin_w, lin_b2d):
    return pl.pallas_call(
        _proj_kernel,
        grid=(pl.cdiv(VOCAB, V_TILE),),
        in_specs=[
            pl.BlockSpec((BATCH, DIM), lambda i: (0, 0)),
            pl.BlockSpec((V_TILE, DIM), lambda i: (i, 0)),
            pl.BlockSpec((1, 1, V_TILE), lambda i: (i, 0, 0)),
        ],
        out_specs=pl.BlockSpec((V_TILE, BATCH), lambda i: (i, 0)),
        out_shape=jax.ShapeDtypeStruct((VOCAB, BATCH), jnp.float32),
        scratch_shapes=[pltpu.VMEM((DIM, BATCH), jnp.float32)],
    )(emb, lin_w, lin_b2d)


def kernel(input_ids, emb_table, lin_w, lin_b):
    emb = _make_gather()(input_ids, emb_table)
    out_t = _project(emb, lin_w, lin_b.reshape(VOCAB // V_TILE, 1, V_TILE))
    return out_t.T
